# baseline (speedup 1.0000x reference)
import functools

import jax
import jax.numpy as jnp
from jax import lax
from jax.experimental import pallas as pl
from jax.experimental.pallas import tpu as pltpu
from jax.experimental.pallas import tpu_sc as plsc

NC = 2
NS = 16
NW = NC * NS
SEQ_CHUNK = 2
WINDOWS = ((0, 80), (80, 80), (160, 40))


def _emb_kernel(tok_hbm, comb_hbm, idx_hbm, cidx_hbm, out_hbm,
                idx_v, cidx_v, tok_v, sem):
  n_seq, seq, emb = out_hbm.shape
  seq_per_tile = n_seq // NW
  rows_chunk = SEQ_CHUNK * seq
  wid = lax.axis_index("s") * NC + lax.axis_index("c")
  seq0 = wid * seq_per_tile

  @pl.loop(0, seq_per_tile, step=SEQ_CHUNK)
  def _chunk(soff):
    sbase = seq0 + soff
    rbase = sbase * seq
    pltpu.sync_copy(idx_hbm.at[pl.ds(rbase, rows_chunk)], idx_v)
    pltpu.sync_copy(cidx_hbm.at[pl.ds(rbase, rows_chunk)], cidx_v)

    copies = []
    for s in range(SEQ_CHUNK):
      for off, ln in WINDOWS:
        copies.append(pltpu.async_copy(
            tok_hbm.at[idx_v.at[pl.ds(s * seq + off, ln)]],
            tok_v.at[s, pl.ds(off, ln)], sem))
    for c in copies:
      c.wait()
    copies = []
    for s in range(SEQ_CHUNK):
      for off, ln in WINDOWS:
        copies.append(pltpu.async_copy(
            comb_hbm.at[cidx_v.at[pl.ds(s * seq + off, ln)]],
            tok_v.at[s, pl.ds(off, ln)], sem, add=True))
    for c in copies:
      c.wait()

    pltpu.sync_copy(tok_v, out_hbm.at[pl.ds(sbase, SEQ_CHUNK)])


def kernel(x, segment_label, token_table, position_table, segment_table):
  batch, seq = x.shape
  emb = token_table.shape[1]
  n = batch * seq

  nseg = segment_table.shape[0]
  combined = (position_table[:seq, None, :]
              + segment_table[None, :, :]).reshape(seq * nseg, emb)

  idx = x.reshape(n).astype(jnp.int32)
  cidx = (jnp.arange(seq, dtype=jnp.int32)[None, :] * nseg
          + segment_label.astype(jnp.int32)).reshape(n)

  mesh = plsc.VectorSubcoreMesh(core_axis_name="c", subcore_axis_name="s",
                                num_cores=NC, num_subcores=NS)
  run = pl.kernel(
      _emb_kernel,
      out_type=jax.ShapeDtypeStruct((batch, seq, emb), jnp.float32),
      mesh=mesh,
      scratch_types=[
          pltpu.VMEM((SEQ_CHUNK * seq,), jnp.int32),
          pltpu.VMEM((SEQ_CHUNK * seq,), jnp.int32),
          pltpu.VMEM((SEQ_CHUNK, seq, emb), jnp.float32),
          pltpu.SemaphoreType.DMA,
      ],
      compiler_params=pltpu.CompilerParams(use_tc_tiling_on_sc=False),
  )
  return run(token_table, combined, idx, cidx)


# speedup vs baseline: 1.1037x; 1.1037x over previous
import functools

import jax
import jax.numpy as jnp
from jax import lax
from jax.experimental import pallas as pl
from jax.experimental.pallas import tpu as pltpu
from jax.experimental.pallas import tpu_sc as plsc

NC = 2
NS = 16
NW = NC * NS
SEQ_CHUNK = 2
WINDOWS = ((0, 80), (80, 80), (160, 40))


def _emb_kernel(tok_hbm, comb_hbm, idx_hbm, cidx_hbm, out_hbm,
                idx_v, cidx_v, tok_v, sem):
  n_seq, seq, _ = out_hbm.shape
  seq_per_tile = n_seq // NW
  rows_chunk = SEQ_CHUNK * seq
  wid = lax.axis_index("s") * NC + lax.axis_index("c")
  seq0 = wid * seq_per_tile

  @pl.loop(0, seq_per_tile, step=SEQ_CHUNK)
  def _chunk(soff):
    sbase = seq0 + soff
    rbase = sbase * seq
    pltpu.sync_copy(idx_hbm.at[pl.ds(rbase, rows_chunk)], idx_v)
    pltpu.sync_copy(cidx_hbm.at[pl.ds(rbase, rows_chunk)], cidx_v)

    copies = []
    for s in range(SEQ_CHUNK):
      for off, ln in WINDOWS:
        copies.append(pltpu.async_copy(
            tok_hbm.at[idx_v.at[pl.ds(s * seq + off, ln)]],
            tok_v.at[s, pl.ds(off, ln)], sem))
    for c in copies:
      c.wait()
    copies = []
    for s in range(SEQ_CHUNK):
      for off, ln in WINDOWS:
        copies.append(pltpu.async_copy(
            comb_hbm.at[cidx_v.at[pl.ds(s * seq + off, ln)]],
            tok_v.at[s, pl.ds(off, ln)], sem, add=True))
    for c in copies:
      c.wait()

    pltpu.sync_copy(tok_v, out_hbm.at[pl.ds(sbase, SEQ_CHUNK)])


def kernel(x, segment_label, token_table, position_table, segment_table):
  batch, seq = x.shape
  emb = token_table.shape[1]
  n = batch * seq

  nseg = segment_table.shape[0]
  combined = (position_table[:seq, None, :]
              + segment_table[None, :, :]).reshape(seq * nseg, emb)

  table128 = jnp.pad(token_table, ((0, 0), (0, 128 - emb)))
  comb128 = jnp.pad(combined, ((0, 0), (0, 128 - emb)))

  idx = x.reshape(n).astype(jnp.int32)
  cidx = (jnp.arange(seq, dtype=jnp.int32)[None, :] * nseg
          + segment_label.astype(jnp.int32)).reshape(n)

  mesh = plsc.VectorSubcoreMesh(core_axis_name="c", subcore_axis_name="s",
                                num_cores=NC, num_subcores=NS)
  run = pl.kernel(
      _emb_kernel,
      out_type=jax.ShapeDtypeStruct((batch, seq, 128), jnp.float32),
      mesh=mesh,
      scratch_types=[
          pltpu.VMEM((SEQ_CHUNK * seq,), jnp.int32),
          pltpu.VMEM((SEQ_CHUNK * seq,), jnp.int32),
          pltpu.VMEM((SEQ_CHUNK, seq, 128), jnp.float32),
          pltpu.SemaphoreType.DMA,
      ],
      compiler_params=pltpu.CompilerParams(use_tc_tiling_on_sc=True),
  )
  out128 = run(table128, comb128, idx, cidx)
  return out128[:, :, :emb]
